# baseline (speedup 1.0000x reference)
import functools

import jax
import jax.numpy as jnp
from jax import lax
from jax.experimental import pallas as pl
from jax.experimental.pallas import tpu as pltpu
from jax.experimental.pallas import tpu_sc as plsc

N = 50000
E = 800000
H = 64
NG = 512
L = 3

NC = 2
NS = 16
NHALF = N // NC
QN = NHALF // 2
TRASH = QN
AR = 12544
PN = 4 * AR
ART = AR // NS
CAP = 13824
NBLK = CAP // 128
NPAIR = NBLK // 2
PBLK = 2000
PNB = (E // NS) // PBLK
OWN = AR // NS
ACR = 800
SUBCAP = 1024
SPB = 2304
NBLK2 = NS * SUBCAP // 128
PPN = 53248
PJ = (PPN // 32) // 128

_MESH = plsc.VectorSubcoreMesh(core_axis_name="c", subcore_axis_name="s")
_CPARAMS = pltpu.CompilerParams(needs_layout_passes=False,
                                use_tc_tiling_on_sc=False)


@functools.partial(
    pl.kernel,
    out_type=[jax.ShapeDtypeStruct((NC, NS, 2, CAP), jnp.int32)] * 3,
    mesh=_MESH,
    scratch_types=[
        pltpu.VMEM((CAP,), jnp.int32),
        pltpu.VMEM((CAP,), jnp.int32),
        pltpu.VMEM((CAP,), jnp.int32),
        pltpu.VMEM((CAP,), jnp.int32),
        pltpu.VMEM((CAP,), jnp.int32),
        pltpu.VMEM((CAP,), jnp.int32),
        pltpu.VMEM((PBLK,), jnp.int32),
        pltpu.VMEM((PBLK,), jnp.int32),
    ],
    compiler_params=_CPARAMS,
)
def _partition(src_hbm, dst_hbm, eid_out, src_out, dst_out,
               eb0, sb0, db0, eb1, sb1, db1, sin, din):
    c = lax.axis_index("c")
    s = lax.axis_index("s")
    base = c * NHALF

    def prefill(i, _):
        off = i * 16
        z = jnp.zeros((16,), jnp.int32)
        t = jnp.full((16,), TRASH, jnp.int32)
        eb0[pl.ds(off, 16)] = z
        sb0[pl.ds(off, 16)] = z
        db0[pl.ds(off, 16)] = t
        eb1[pl.ds(off, 16)] = z
        sb1[pl.ds(off, 16)] = z
        db1[pl.ds(off, 16)] = t
        return 0

    lax.fori_loop(0, CAP // 16, prefill, 0)

    def blk(b, curs):
        pltpu.sync_copy(src_hbm.at[s, b], sin)
        pltpu.sync_copy(dst_hbm.at[s, b], din)

        def grp(g, curs):
            cur0, cur1 = curs
            off = g * 16
            dv = din[pl.ds(off, 16)]
            sv = sin[pl.ds(off, 16)]
            eidv = s * (E // NS) + b * PBLK + off + lax.iota(jnp.int32, 16)
            dloc = dv - base
            m = (dloc >= 0) & (dloc < NHALF)
            hi = dloc >= QN
            m0 = m & jnp.logical_not(hi)
            m1 = m & hi
            dq = dloc - jnp.where(hi, QN, 0)
            cs0 = plsc.cumsum(m0.astype(jnp.int32))
            pos0 = cur0 + cs0 - 1
            plsc.store_scatter(db0, [pos0], dq, mask=m0)
            plsc.store_scatter(sb0, [pos0], sv, mask=m0)
            plsc.store_scatter(eb0, [pos0], eidv, mask=m0)
            cs1 = plsc.cumsum(m1.astype(jnp.int32))
            pos1 = cur1 + cs1 - 1
            plsc.store_scatter(db1, [pos1], dq, mask=m1)
            plsc.store_scatter(sb1, [pos1], sv, mask=m1)
            plsc.store_scatter(eb1, [pos1], eidv, mask=m1)
            return (jnp.minimum(cur0 + jnp.max(cs0), CAP - 16),
                    jnp.minimum(cur1 + jnp.max(cs1), CAP - 16))

        return lax.fori_loop(0, PBLK // 16, grp, curs)

    lax.fori_loop(0, PNB, blk, (jnp.int32(0), jnp.int32(0)))
    pltpu.sync_copy(eb0, eid_out.at[c, s, 0])
    pltpu.sync_copy(sb0, src_out.at[c, s, 0])
    pltpu.sync_copy(db0, dst_out.at[c, s, 0])
    pltpu.sync_copy(eb1, eid_out.at[c, s, 1])
    pltpu.sync_copy(sb1, src_out.at[c, s, 1])
    pltpu.sync_copy(db1, dst_out.at[c, s, 1])


@functools.partial(
    pl.kernel,
    out_type=[jax.ShapeDtypeStruct((NC, 2, NS, NS, SUBCAP), jnp.int32)] * 3,
    mesh=_MESH,
    scratch_types=[
        pltpu.VMEM((NS, SUBCAP), jnp.int32),
        pltpu.VMEM((NS, SUBCAP), jnp.int32),
        pltpu.VMEM((NS, SUBCAP), jnp.int32),
        pltpu.VMEM((SPB,), jnp.int32),
        pltpu.VMEM((SPB,), jnp.int32),
        pltpu.VMEM((SPB,), jnp.int32),
    ],
    compiler_params=_CPARAMS,
)
def _subpart(eid_hbm, src_hbm, dst_hbm, eid_out, src_out, dst_out,
             eb, sb, db, ein, sin, din):
    c = lax.axis_index("c")
    s = lax.axis_index("s")

    for q in range(2):
        def prefill(i, _):
            off = i * 16
            z = jnp.zeros((16,), jnp.int32)
            t = jnp.full((16,), OWN, jnp.int32)
            row = off // SUBCAP
            col = off % SUBCAP
            eb[row, pl.ds(col, 16)] = z
            sb[row, pl.ds(col, 16)] = z
            db[row, pl.ds(col, 16)] = t
            return 0

        lax.fori_loop(0, NS * SUBCAP // 16, prefill, 0)

        def blk(b, curs):
            pltpu.sync_copy(eid_hbm.at[c, s, q, pl.ds(b * SPB, SPB)], ein)
            pltpu.sync_copy(src_hbm.at[c, s, q, pl.ds(b * SPB, SPB)], sin)
            pltpu.sync_copy(dst_hbm.at[c, s, q, pl.ds(b * SPB, SPB)], din)

            def grp(g, curs):
                off = g * 16
                dv = din[pl.ds(off, 16)]
                sv = sin[pl.ds(off, 16)]
                ev = ein[pl.ds(off, 16)]
                o = dv // OWN
                dl = dv - o * OWN
                newcurs = []
                for ow in range(NS):
                    m = o == ow
                    cs = plsc.cumsum(m.astype(jnp.int32))
                    pos = curs[ow] + cs - 1
                    plsc.store_scatter(db.at[ow], [pos], dl, mask=m)
                    plsc.store_scatter(sb.at[ow], [pos], sv, mask=m)
                    plsc.store_scatter(eb.at[ow], [pos], ev, mask=m)
                    newcurs.append(
                        jnp.minimum(curs[ow] + jnp.max(cs), SUBCAP - 16))
                return tuple(newcurs)

            return lax.fori_loop(0, SPB // 16, grp, curs)

        lax.fori_loop(0, CAP // SPB, blk,
                      tuple(jnp.int32(0) for _ in range(NS)))
        pltpu.sync_copy(eb, eid_out.at[c, q, :, s])
        pltpu.sync_copy(sb, src_out.at[c, q, :, s])
        pltpu.sync_copy(db, dst_out.at[c, q, :, s])


@functools.partial(
    pl.kernel,
    out_type=[jax.ShapeDtypeStruct((NC, 2, NS, NS * SUBCAP), jnp.int32)] * 3,
    mesh=_MESH,
    scratch_types=[
        pltpu.VMEM((NS * SUBCAP,), jnp.int32),
        pltpu.VMEM((NS * SUBCAP,), jnp.int32),
        pltpu.VMEM((NS * SUBCAP,), jnp.int32),
        pltpu.VMEM((NS * SUBCAP,), jnp.int32),
        pltpu.VMEM((NS * SUBCAP,), jnp.int32),
        pltpu.VMEM((NS * SUBCAP,), jnp.int32),
        pltpu.SMEM((ACR,), jnp.int32),
    ],
    compiler_params=_CPARAMS,
)
def _sortlists(eid_hbm, src_hbm, dst_hbm, eid_out, src_out, dst_out,
               ei, si, di, eo, so, do, pos):
    c = lax.axis_index("c")
    s = lax.axis_index("s")
    lane0 = lax.iota(jnp.int32, 16) == 0
    ne = NS * SUBCAP

    for q in range(2):
        pltpu.sync_copy(eid_hbm.at[c, q, s], ei)
        pltpu.sync_copy(src_hbm.at[c, q, s], si)
        pltpu.sync_copy(dst_hbm.at[c, q, s], di)

        def zcnt(i, _):
            pos[i] = 0
            return 0

        lax.fori_loop(0, ACR, zcnt, 0)

        def count(g, _):
            dv = di[pl.ds(g * 16, 16)]
            for j in range(16):
                d = dv[j]
                pos[d] = pos[d] + 1
            return 0

        lax.fori_loop(0, ne // 16, count, 0)

        def prefix(i, tot):
            cnt = pos[i]
            pos[i] = tot
            return tot + cnt

        lax.fori_loop(0, ACR, prefix, jnp.int32(0))

        def permute(g, _):
            dv = di[pl.ds(g * 16, 16)]
            sv = si[pl.ds(g * 16, 16)]
            ev = ei[pl.ds(g * 16, 16)]
            for j in range(16):
                d = dv[j]
                p = pos[d]
                pos[d] = p + 1
                pv = jnp.full((16,), p, jnp.int32)
                gd = jnp.where(d == OWN, AR, s * OWN + d)
                plsc.store_scatter(do, [pv],
                                   jnp.full((16,), gd, jnp.int32), mask=lane0)
                plsc.store_scatter(so, [pv],
                                   jnp.full((16,), sv[j], jnp.int32),
                                   mask=lane0)
                plsc.store_scatter(eo, [pv],
                                   jnp.full((16,), ev[j], jnp.int32),
                                   mask=lane0)
            return 0

        lax.fori_loop(0, ne // 16, permute, 0)
        pltpu.sync_copy(eo, eid_out.at[c, q, s])
        pltpu.sync_copy(so, src_out.at[c, q, s])
        pltpu.sync_copy(do, dst_out.at[c, q, s])


@functools.partial(
    pl.kernel,
    out_type=jax.ShapeDtypeStruct((NC, 2, NS, NS * SUBCAP, H), jnp.float32),
    mesh=_MESH,
    scratch_types=[
        pltpu.VMEM((2, 1, 128), jnp.int32),
        pltpu.VMEM((2, 128, H), jnp.float32),
        pltpu.SemaphoreType.DMA,
        pltpu.SemaphoreType.DMA,
        pltpu.SemaphoreType.DMA,
        pltpu.SemaphoreType.DMA,
    ],
    compiler_params=_CPARAMS,
)
def _perme(e_hbm, eid_hbm, ep_out, eidx, eb, si0, si1, se0, se1):
    c = lax.axis_index("c")
    s = lax.axis_index("s")
    sis = (si0, si1)
    ses = (se0, se1)

    for q in range(2):
        def fire_idx(b, p):
            pltpu.async_copy(eid_hbm.at[c, q, s, pl.ds(b * 128, 128)],
                             eidx.at[p, 0], sis[p])

        def wait_idx(b, p):
            pltpu.make_async_copy(eid_hbm.at[c, q, s, pl.ds(b * 128, 128)],
                                  eidx.at[p, 0], sis[p]).wait()

        def fire_gather(p):
            pltpu.async_copy(e_hbm.at[eidx.at[p, 0]], eb.at[p], ses[p])

        def wait_gather(p):
            pltpu.make_async_copy(e_hbm.at[eidx.at[p, 0]], eb.at[p],
                                  ses[p]).wait()

        fire_idx(0, 0)
        wait_idx(0, 0)
        fire_idx(1, 1)
        fire_gather(0)

        def pair(pp, _):
            b0 = pp * 2
            b1 = b0 + 1
            for p, b in ((0, b0), (1, b1)):
                wait_gather(p)

                @pl.when(b + 1 < NBLK2)
                def _():
                    wait_idx(b + 1, 1 - p)
                    fire_gather(1 - p)

                pltpu.sync_copy(eb.at[p],
                                ep_out.at[c, q, s, pl.ds(b * 128, 128)])

                @pl.when(b + 2 < NBLK2)
                def _():
                    fire_idx(b + 2, p)

            return 0

        lax.fori_loop(0, NBLK2 // 2, pair, 0)


@functools.partial(
    pl.kernel,
    out_type=jax.ShapeDtypeStruct((NC, 2, AR, H), jnp.float32),
    mesh=_MESH,
    scratch_types=[
        pltpu.VMEM_SHARED((AR + 64, H), jnp.float32),
        pltpu.VMEM((4, 1, 128), jnp.int32),
        pltpu.VMEM((4, 1, 128), jnp.int32),
        pltpu.VMEM((4, 128, H), jnp.float32),
        pltpu.VMEM((4, 128, H), jnp.float32),
        pltpu.VMEM((OWN // 4, H), jnp.float32),
        pltpu.SemaphoreType.DMA,
        pltpu.SemaphoreType.DMA,
        pltpu.SemaphoreType.DMA,
        pltpu.SemaphoreType.DMA,
        pltpu.SemaphoreType.DMA,
        pltpu.SemaphoreType.DMA,
        pltpu.SemaphoreType.DMA,
        pltpu.SemaphoreType.DMA,
        pltpu.SemaphoreType.DMA,
        pltpu.SemaphoreType.DMA,
        pltpu.SemaphoreType.DMA,
        pltpu.SemaphoreType.DMA,
    ],
    compiler_params=_CPARAMS,
)
def _layer_agg(h_hbm, ep_hbm, src_hbm, dst_hbm, agg_out,
               acc, sidx, didx, hb, ebf, zb,
               si0, si1, si2, si3, sh0, sh1, sh2, sh3, se0, se1, se2, se3):
    c = lax.axis_index("c")
    s = lax.axis_index("s")
    sis = (si0, si1, si2, si3)
    shs = (sh0, sh1, sh2, sh3)
    ses = (se0, se1, se2, se3)

    def zrow(r, _):
        for k in range(H // 16):
            zb[r, pl.ds(k * 16, 16)] = jnp.zeros((16,), jnp.float32)
        return 0

    lax.fori_loop(0, OWN // 4, zrow, 0)

    for q in range(2):
        def idx_trips(b, p):
            return ((src_hbm.at[c, q, s, pl.ds(b * 128, 128)], sidx.at[p, 0]),
                    (dst_hbm.at[c, q, s, pl.ds(b * 128, 128)], didx.at[p, 0]))

        def fire_idx(b, p):
            for sr, dr in idx_trips(b, p):
                pltpu.async_copy(sr, dr, sis[p])

        def wait_idx(b, p):
            for sr, dr in idx_trips(b, p):
                pltpu.make_async_copy(sr, dr, sis[p]).wait()

        def fire_gather(b, p):
            pltpu.async_copy(h_hbm.at[sidx.at[p, 0]], hb.at[p], shs[p])
            pltpu.async_copy(ep_hbm.at[c, q, s, pl.ds(b * 128, 128)],
                             ebf.at[p], ses[p])

        def wait_gather(b, p):
            pltpu.make_async_copy(h_hbm.at[sidx.at[p, 0]], hb.at[p],
                                  shs[p]).wait()
            pltpu.make_async_copy(ep_hbm.at[c, q, s, pl.ds(b * 128, 128)],
                                  ebf.at[p], ses[p]).wait()

        def compute(p):
            def rr(r, _):
                for k in range(H // 16):
                    v = (hb[p, r, pl.ds(k * 16, 16)]
                         + ebf[p, r, pl.ds(k * 16, 16)])
                    hb[p, r, pl.ds(k * 16, 16)] = jnp.maximum(v, 0.0)
                return 0

            lax.fori_loop(0, 128, rr, 0)

        def scatter(p):
            pltpu.sync_copy(hb.at[p], acc.at[didx.at[p, 0]], add=True)

        for j in range(4):
            pltpu.sync_copy(zb, acc.at[pl.ds(s * OWN + j * (OWN // 4),
                                             OWN // 4)])

        fire_idx(0, 0)
        wait_idx(0, 0)
        for b0 in (1, 2, 3):
            fire_idx(b0, b0)
        fire_gather(0, 0)
        for b0 in (1, 2):
            wait_idx(b0, b0)
            fire_gather(b0, b0)

        def quad(qq, _):
            base = qq * 4
            for p in range(4):
                b = base + p
                wait_gather(b, p)
                compute(p)
                scatter(p)

                @pl.when(b + 4 < NBLK2)
                def _():
                    fire_idx(b + 4, p)

                @pl.when(b + 3 < NBLK2)
                def _():
                    wait_idx(b + 3, (p + 3) % 4)
                    fire_gather(b + 3, (p + 3) % 4)

            return 0

        lax.fori_loop(0, NBLK2 // 4, quad, 0)
        pltpu.sync_copy(acc.at[pl.ds(s * OWN, OWN)],
                        agg_out.at[c, q, pl.ds(s * OWN, OWN)])


@functools.partial(
    pl.kernel,
    out_type=[jax.ShapeDtypeStruct((NC, 528, H), jnp.float32),
              jax.ShapeDtypeStruct((NC, 528, 16), jnp.float32)],
    mesh=_MESH,
    scratch_types=[
        pltpu.VMEM_SHARED((528, H), jnp.float32),
        pltpu.VMEM_SHARED((528, 16), jnp.float32),
        pltpu.VMEM((PJ, 128, H), jnp.float32),
        pltpu.VMEM((PJ, 128), jnp.int32),
        pltpu.VMEM((128, 16), jnp.float32),
        pltpu.VMEM((33, H), jnp.float32),
    ],
    compiler_params=_CPARAMS,
)
def _pool(h_hbm, b_hbm, sum_out, cnt_out, ps, pc, hblk, bidx, ones, zb):
    c = lax.axis_index("c")
    s = lax.axis_index("s")
    wid = s * NC + c

    def zrow(r, _):
        for k in range(H // 16):
            zb[r, pl.ds(k * 16, 16)] = jnp.zeros((16,), jnp.float32)
        return 0

    lax.fori_loop(0, 33, zrow, 0)

    def orow(r, _):
        ones[r, pl.ds(0, 16)] = jnp.ones((16,), jnp.float32)
        return 0

    lax.fori_loop(0, 128, orow, 0)
    pltpu.sync_copy(zb, ps.at[pl.ds(s * 33, 33)])
    pltpu.sync_copy(zb.at[pl.ds(0, 33), pl.ds(0, 16)], pc.at[pl.ds(s * 33, 33)])
    plsc.subcore_barrier()

    pltpu.sync_copy(h_hbm.at[wid], hblk)
    pltpu.sync_copy(b_hbm.at[wid], bidx)
    for j in range(PJ):
        pltpu.sync_copy(hblk.at[j], ps.at[bidx.at[j]], add=True)
        pltpu.sync_copy(ones, pc.at[bidx.at[j]], add=True)
    plsc.subcore_barrier()
    pltpu.sync_copy(ps.at[pl.ds(s * 33, 33)], sum_out.at[c, pl.ds(s * 33, 33)])
    pltpu.sync_copy(pc.at[pl.ds(s * 33, 33)], cnt_out.at[c, pl.ds(s * 33, 33)])


def _enc_body(a_ref, w_ref, b_ref, o_ref):
    o_ref[...] = jnp.dot(a_ref[...], w_ref[...],
                         preferred_element_type=jnp.float32) + b_ref[...]


def _encode(a, w, b, blk):
    n = a.shape[0]
    k = a.shape[1]
    return pl.pallas_call(
        _enc_body,
        grid=(n // blk,),
        in_specs=[
            pl.BlockSpec((blk, k), lambda i: (i, 0)),
            pl.BlockSpec((k, H), lambda i: (0, 0)),
            pl.BlockSpec((1, H), lambda i: (0, 0)),
        ],
        out_specs=pl.BlockSpec((blk, H), lambda i: (i, 0)),
        out_shape=jax.ShapeDtypeStruct((n, H), jnp.float32),
    )(a, w, b.reshape(1, H))


def _mlp_body(h_ref, agg_ref, eps_ref, w1_ref, b1_ref, w2_ref, b2_ref, o_ref):
    h = h_ref[...]
    z = eps_ref[0, 0] * h + agg_ref[...]
    t = jnp.maximum(jnp.dot(z, w1_ref[...],
                            preferred_element_type=jnp.float32) + b1_ref[...],
                    0.0)
    z2 = jnp.dot(t, w2_ref[...], preferred_element_type=jnp.float32) + b2_ref[...]
    o_ref[...] = h + jnp.maximum(z2, 0.0)


def _mlp(h, agg, epsi, w1, b1, w2, b2):
    blk = 1568
    nb = PN // blk
    return pl.pallas_call(
        _mlp_body,
        grid=(nb,),
        in_specs=[
            pl.BlockSpec((blk, H), lambda i: (i, 0)),
            pl.BlockSpec((blk, H), lambda i: (i, 0)),
            pl.BlockSpec((1, 1), lambda i: (0, 0)),
            pl.BlockSpec((H, H), lambda i: (0, 0)),
            pl.BlockSpec((1, H), lambda i: (0, 0)),
            pl.BlockSpec((H, H), lambda i: (0, 0)),
            pl.BlockSpec((1, H), lambda i: (0, 0)),
        ],
        out_specs=pl.BlockSpec((blk, H), lambda i: (i, 0)),
        out_shape=jax.ShapeDtypeStruct((PN, H), jnp.float32),
    )(h, agg, epsi.reshape(1, 1), w1, b1.reshape(1, H), w2, b2.reshape(1, H))


def _head_body(ps_ref, pc_ref, w1_ref, b1_ref, w2_ref, b2_ref, g_ref, bb_ref,
               o_ref):
    sums = ps_ref[0, :NG, :] + ps_ref[1, :NG, :]
    counts = pc_ref[0, :NG, 0:1] + pc_ref[1, :NG, 0:1]
    xg = sums / jnp.maximum(counts, 1.0)
    t = jnp.maximum(jnp.dot(xg, w1_ref[...],
                            preferred_element_type=jnp.float32) + b1_ref[...],
                    0.0)
    o = jnp.dot(t, w2_ref[...], preferred_element_type=jnp.float32) + b2_ref[...]
    mu = jnp.mean(o, axis=-1, keepdims=True)
    var = jnp.mean((o - mu) * (o - mu), axis=-1, keepdims=True)
    o_ref[...] = (o - mu) / jnp.sqrt(var + 1e-5) * g_ref[...] + bb_ref[...]


def _head(psum, pcnt, w1, b1, w2, b2, g, bb):
    out_dim = w2.shape[1]
    return pl.pallas_call(
        _head_body,
        out_shape=jax.ShapeDtypeStruct((NG, out_dim), jnp.float32),
    )(psum, pcnt, w1, b1.reshape(1, H), w2, b2.reshape(1, out_dim),
      g.reshape(1, out_dim), bb.reshape(1, out_dim))


def kernel(x, edge_index, edge_attr, batch, W_atom, b_atom, W_edge, b_edge,
           eps, W1, b1, W2, b2, Wo1, bo1, Wo2, bo2, ln_g, ln_b):
    src0 = edge_index[0].astype(jnp.int32)
    src_p = (src0 + (AR - QN) * (src0 // QN)).reshape(NS, PNB, PBLK)
    dst = edge_index[1].astype(jnp.int32).reshape(NS, PNB, PBLK)
    eidL, srcL, dstL = _partition(src_p, dst)
    eidS, srcS, dstS = _subpart(eidL, srcL, dstL)
    eidS, srcS, dstS = _sortlists(eidS.reshape(NC, 2, NS, NS * SUBCAP),
                                  srcS.reshape(NC, 2, NS, NS * SUBCAP),
                                  dstS.reshape(NC, 2, NS, NS * SUBCAP))

    ad = x.shape[1]
    xp = jnp.pad(x.reshape(4, QN, ad),
                 ((0, 0), (0, AR - QN), (0, 0))).reshape(PN, ad)
    h = _encode(xp, W_atom, b_atom, 1568)
    e = _encode(edge_attr, W_edge, b_edge, 8000)
    ep = _perme(e, eidS)

    for i in range(L):
        aggp = _layer_agg(h, ep, srcS, dstS)
        h = _mlp(h, aggp.reshape(PN, H), 1.0 + eps[i],
                 W1[i], b1[i], W2[i], b2[i])

    bpq = jnp.pad(batch.astype(jnp.int32).reshape(4, QN),
                  ((0, 0), (0, AR - QN)), constant_values=NG).reshape(PN)
    hp = jnp.pad(h, ((0, PPN - PN), (0, 0))).reshape(32, PJ, 128, H)
    bp = jnp.pad(bpq, (0, PPN - PN), constant_values=NG).reshape(32, PJ, 128)
    psum, pcnt = _pool(hp, bp)
    return _head(psum, pcnt, Wo1, bo1, Wo2, bo2, ln_g, ln_b)


# speedup vs baseline: 2.9969x; 2.9969x over previous
import functools

import jax
import jax.numpy as jnp
from jax import lax
from jax.experimental import pallas as pl
from jax.experimental.pallas import tpu as pltpu
from jax.experimental.pallas import tpu_sc as plsc

N = 50000
E = 800000
H = 64
NG = 512
L = 3

NC = 2
NS = 16
NHALF = N // NC
QN = NHALF // 2
TRASH = QN
AR = 12544
PN = 4 * AR
ART = AR // NS
CAP = 13824
NBLK = CAP // 128
NPAIR = NBLK // 2
PBLK = 2000
PNB = (E // NS) // PBLK
OWN = AR // NS
ACR = 800
SUBCAP = 1024
SPB = 2304
NBLK2 = NS * SUBCAP // 128
PPN = 53248
PJ = (PPN // 32) // 128

_MESH = plsc.VectorSubcoreMesh(core_axis_name="c", subcore_axis_name="s")
_CPARAMS = pltpu.CompilerParams(needs_layout_passes=False,
                                use_tc_tiling_on_sc=False)


@functools.partial(
    pl.kernel,
    out_type=[jax.ShapeDtypeStruct((NC, NS, 2, CAP), jnp.int32)] * 3,
    mesh=_MESH,
    scratch_types=[
        pltpu.VMEM((CAP,), jnp.int32),
        pltpu.VMEM((CAP,), jnp.int32),
        pltpu.VMEM((CAP,), jnp.int32),
        pltpu.VMEM((CAP,), jnp.int32),
        pltpu.VMEM((CAP,), jnp.int32),
        pltpu.VMEM((CAP,), jnp.int32),
        pltpu.VMEM((PBLK,), jnp.int32),
        pltpu.VMEM((PBLK,), jnp.int32),
    ],
    compiler_params=_CPARAMS,
)
def _partition(src_hbm, dst_hbm, eid_out, src_out, dst_out,
               eb0, sb0, db0, eb1, sb1, db1, sin, din):
    c = lax.axis_index("c")
    s = lax.axis_index("s")
    base = c * NHALF

    def prefill(i, _):
        off = i * 16
        z = jnp.zeros((16,), jnp.int32)
        t = jnp.full((16,), TRASH, jnp.int32)
        eb0[pl.ds(off, 16)] = z
        sb0[pl.ds(off, 16)] = z
        db0[pl.ds(off, 16)] = t
        eb1[pl.ds(off, 16)] = z
        sb1[pl.ds(off, 16)] = z
        db1[pl.ds(off, 16)] = t
        return 0

    lax.fori_loop(0, CAP // 16, prefill, 0)

    def blk(b, curs):
        pltpu.sync_copy(src_hbm.at[s, b], sin)
        pltpu.sync_copy(dst_hbm.at[s, b], din)

        def grp(g, curs):
            cur0, cur1 = curs
            off = g * 16
            dv = din[pl.ds(off, 16)]
            sv = sin[pl.ds(off, 16)]
            eidv = s * (E // NS) + b * PBLK + off + lax.iota(jnp.int32, 16)
            dloc = dv - base
            m = (dloc >= 0) & (dloc < NHALF)
            hi = dloc >= QN
            m0 = m & jnp.logical_not(hi)
            m1 = m & hi
            dq = dloc - jnp.where(hi, QN, 0)
            cs0 = plsc.cumsum(m0.astype(jnp.int32))
            pos0 = cur0 + cs0 - 1
            plsc.store_scatter(db0, [pos0], dq, mask=m0)
            plsc.store_scatter(sb0, [pos0], sv, mask=m0)
            plsc.store_scatter(eb0, [pos0], eidv, mask=m0)
            cs1 = plsc.cumsum(m1.astype(jnp.int32))
            pos1 = cur1 + cs1 - 1
            plsc.store_scatter(db1, [pos1], dq, mask=m1)
            plsc.store_scatter(sb1, [pos1], sv, mask=m1)
            plsc.store_scatter(eb1, [pos1], eidv, mask=m1)
            return (jnp.minimum(cur0 + jnp.max(cs0), CAP - 16),
                    jnp.minimum(cur1 + jnp.max(cs1), CAP - 16))

        return lax.fori_loop(0, PBLK // 16, grp, curs)

    lax.fori_loop(0, PNB, blk, (jnp.int32(0), jnp.int32(0)))
    pltpu.sync_copy(eb0, eid_out.at[c, s, 0])
    pltpu.sync_copy(sb0, src_out.at[c, s, 0])
    pltpu.sync_copy(db0, dst_out.at[c, s, 0])
    pltpu.sync_copy(eb1, eid_out.at[c, s, 1])
    pltpu.sync_copy(sb1, src_out.at[c, s, 1])
    pltpu.sync_copy(db1, dst_out.at[c, s, 1])


@functools.partial(
    pl.kernel,
    out_type=jax.ShapeDtypeStruct((NC, 2, AR, H), jnp.float32),
    mesh=_MESH,
    scratch_types=[
        pltpu.VMEM_SHARED((AR, H), jnp.float32),
        pltpu.VMEM((2, 1, 128), jnp.int32),
        pltpu.VMEM((2, 1, 128), jnp.int32),
        pltpu.VMEM((2, 1, 128), jnp.int32),
        pltpu.VMEM((2, 128, H), jnp.float32),
        pltpu.VMEM((2, 128, H), jnp.float32),
        pltpu.VMEM((ART // 4, H), jnp.float32),
        pltpu.SemaphoreType.DMA,
        pltpu.SemaphoreType.DMA,
        pltpu.SemaphoreType.DMA,
        pltpu.SemaphoreType.DMA,
        pltpu.SemaphoreType.DMA,
        pltpu.SemaphoreType.DMA,
    ],
    compiler_params=_CPARAMS,
)
def _layer_agg(h_hbm, e_hbm, eid_hbm, src_hbm, dst_hbm, agg_out,
               acc, eidx, sidx, didx, hb, ebf, zb,
               si0, si1, sh0, sh1, se0, se1):
    c = lax.axis_index("c")
    s = lax.axis_index("s")
    sis = (si0, si1)
    shs = (sh0, sh1)
    ses = (se0, se1)

    def zrow(r, _):
        for k in range(H // 16):
            zb[r, pl.ds(k * 16, 16)] = jnp.zeros((16,), jnp.float32)
        return 0

    lax.fori_loop(0, ART // 4, zrow, 0)

    for q in range(2):
        def idx_trips(b, p):
            return ((eid_hbm.at[c, s, q, pl.ds(b * 128, 128)], eidx.at[p, 0]),
                    (src_hbm.at[c, s, q, pl.ds(b * 128, 128)], sidx.at[p, 0]),
                    (dst_hbm.at[c, s, q, pl.ds(b * 128, 128)], didx.at[p, 0]))

        def fire_idx(b, p):
            for sr, dr in idx_trips(b, p):
                pltpu.async_copy(sr, dr, sis[p])

        def wait_idx(b, p):
            for sr, dr in idx_trips(b, p):
                pltpu.make_async_copy(sr, dr, sis[p]).wait()

        def fire_gather(b, p):
            del b
            pltpu.async_copy(h_hbm.at[sidx.at[p, 0]], hb.at[p], shs[p])
            pltpu.async_copy(e_hbm.at[eidx.at[p, 0]], ebf.at[p], ses[p])

        def wait_gather(p):
            pltpu.make_async_copy(h_hbm.at[sidx.at[p, 0]], hb.at[p],
                                  shs[p]).wait()
            pltpu.make_async_copy(e_hbm.at[eidx.at[p, 0]], ebf.at[p],
                                  ses[p]).wait()

        def compute(p):
            def rr(r, _):
                for k in range(H // 16):
                    v = (hb[p, r, pl.ds(k * 16, 16)]
                         + ebf[p, r, pl.ds(k * 16, 16)])
                    hb[p, r, pl.ds(k * 16, 16)] = jnp.maximum(v, 0.0)
                return 0

            lax.fori_loop(0, 128, rr, 0)

        def scatter(p):
            pltpu.sync_copy(hb.at[p], acc.at[didx.at[p, 0]], add=True)

        for j in range(4):
            pltpu.sync_copy(
                zb, acc.at[pl.ds(s * ART + j * (ART // 4), ART // 4)])
        plsc.subcore_barrier()

        fire_idx(0, 0)
        wait_idx(0, 0)
        fire_idx(1, 1)
        fire_gather(0, 0)

        def pair(pp, _):
            b0 = pp * 2
            b1 = b0 + 1
            for p, b in ((0, b0), (1, b1)):
                wait_gather(p)
                compute(p)

                @pl.when(b + 1 < NBLK)
                def _():
                    wait_idx(b + 1, 1 - p)
                    fire_gather(b + 1, 1 - p)

                scatter(p)

                @pl.when(b + 2 < NBLK)
                def _():
                    fire_idx(b + 2, p)

            return 0

        lax.fori_loop(0, NPAIR, pair, 0)
        plsc.subcore_barrier()
        pltpu.sync_copy(acc.at[pl.ds(s * ART, ART)],
                        agg_out.at[c, q, pl.ds(s * ART, ART)])


@functools.partial(
    pl.kernel,
    out_type=[jax.ShapeDtypeStruct((NC, 528, H), jnp.float32),
              jax.ShapeDtypeStruct((NC, 528, 16), jnp.float32)],
    mesh=_MESH,
    scratch_types=[
        pltpu.VMEM_SHARED((528, H), jnp.float32),
        pltpu.VMEM_SHARED((528, 16), jnp.float32),
        pltpu.VMEM((PJ, 128, H), jnp.float32),
        pltpu.VMEM((PJ, 128), jnp.int32),
        pltpu.VMEM((128, 16), jnp.float32),
        pltpu.VMEM((33, H), jnp.float32),
    ],
    compiler_params=_CPARAMS,
)
def _pool(h_hbm, b_hbm, sum_out, cnt_out, ps, pc, hblk, bidx, ones, zb):
    c = lax.axis_index("c")
    s = lax.axis_index("s")
    wid = s * NC + c

    def zrow(r, _):
        for k in range(H // 16):
            zb[r, pl.ds(k * 16, 16)] = jnp.zeros((16,), jnp.float32)
        return 0

    lax.fori_loop(0, 33, zrow, 0)

    def orow(r, _):
        ones[r, pl.ds(0, 16)] = jnp.ones((16,), jnp.float32)
        return 0

    lax.fori_loop(0, 128, orow, 0)
    pltpu.sync_copy(zb, ps.at[pl.ds(s * 33, 33)])
    pltpu.sync_copy(zb.at[pl.ds(0, 33), pl.ds(0, 16)], pc.at[pl.ds(s * 33, 33)])
    plsc.subcore_barrier()

    pltpu.sync_copy(h_hbm.at[wid], hblk)
    pltpu.sync_copy(b_hbm.at[wid], bidx)
    for j in range(PJ):
        pltpu.sync_copy(hblk.at[j], ps.at[bidx.at[j]], add=True)
        pltpu.sync_copy(ones, pc.at[bidx.at[j]], add=True)
    plsc.subcore_barrier()
    pltpu.sync_copy(ps.at[pl.ds(s * 33, 33)], sum_out.at[c, pl.ds(s * 33, 33)])
    pltpu.sync_copy(pc.at[pl.ds(s * 33, 33)], cnt_out.at[c, pl.ds(s * 33, 33)])


def _enc_body(a_ref, w_ref, b_ref, o_ref):
    o_ref[...] = jnp.dot(a_ref[...], w_ref[...],
                         preferred_element_type=jnp.float32) + b_ref[...]


def _encode(a, w, b, blk):
    n = a.shape[0]
    k = a.shape[1]
    return pl.pallas_call(
        _enc_body,
        grid=(n // blk,),
        in_specs=[
            pl.BlockSpec((blk, k), lambda i: (i, 0)),
            pl.BlockSpec((k, H), lambda i: (0, 0)),
            pl.BlockSpec((1, H), lambda i: (0, 0)),
        ],
        out_specs=pl.BlockSpec((blk, H), lambda i: (i, 0)),
        out_shape=jax.ShapeDtypeStruct((n, H), jnp.float32),
    )(a, w, b.reshape(1, H))


def _mlp_body(h_ref, agg_ref, eps_ref, w1_ref, b1_ref, w2_ref, b2_ref, o_ref):
    h = h_ref[...]
    z = eps_ref[0, 0] * h + agg_ref[...]
    t = jnp.maximum(jnp.dot(z, w1_ref[...],
                            preferred_element_type=jnp.float32) + b1_ref[...],
                    0.0)
    z2 = jnp.dot(t, w2_ref[...], preferred_element_type=jnp.float32) + b2_ref[...]
    o_ref[...] = h + jnp.maximum(z2, 0.0)


def _mlp(h, agg, epsi, w1, b1, w2, b2):
    blk = 1568
    nb = PN // blk
    return pl.pallas_call(
        _mlp_body,
        grid=(nb,),
        in_specs=[
            pl.BlockSpec((blk, H), lambda i: (i, 0)),
            pl.BlockSpec((blk, H), lambda i: (i, 0)),
            pl.BlockSpec((1, 1), lambda i: (0, 0)),
            pl.BlockSpec((H, H), lambda i: (0, 0)),
            pl.BlockSpec((1, H), lambda i: (0, 0)),
            pl.BlockSpec((H, H), lambda i: (0, 0)),
            pl.BlockSpec((1, H), lambda i: (0, 0)),
        ],
        out_specs=pl.BlockSpec((blk, H), lambda i: (i, 0)),
        out_shape=jax.ShapeDtypeStruct((PN, H), jnp.float32),
    )(h, agg, epsi.reshape(1, 1), w1, b1.reshape(1, H), w2, b2.reshape(1, H))


def _head_body(ps_ref, pc_ref, w1_ref, b1_ref, w2_ref, b2_ref, g_ref, bb_ref,
               o_ref):
    sums = ps_ref[0, :NG, :] + ps_ref[1, :NG, :]
    counts = pc_ref[0, :NG, 0:1] + pc_ref[1, :NG, 0:1]
    xg = sums / jnp.maximum(counts, 1.0)
    t = jnp.maximum(jnp.dot(xg, w1_ref[...],
                            preferred_element_type=jnp.float32) + b1_ref[...],
                    0.0)
    o = jnp.dot(t, w2_ref[...], preferred_element_type=jnp.float32) + b2_ref[...]
    mu = jnp.mean(o, axis=-1, keepdims=True)
    var = jnp.mean((o - mu) * (o - mu), axis=-1, keepdims=True)
    o_ref[...] = (o - mu) / jnp.sqrt(var + 1e-5) * g_ref[...] + bb_ref[...]


def _head(psum, pcnt, w1, b1, w2, b2, g, bb):
    out_dim = w2.shape[1]
    return pl.pallas_call(
        _head_body,
        out_shape=jax.ShapeDtypeStruct((NG, out_dim), jnp.float32),
    )(psum, pcnt, w1, b1.reshape(1, H), w2, b2.reshape(1, out_dim),
      g.reshape(1, out_dim), bb.reshape(1, out_dim))


def kernel(x, edge_index, edge_attr, batch, W_atom, b_atom, W_edge, b_edge,
           eps, W1, b1, W2, b2, Wo1, bo1, Wo2, bo2, ln_g, ln_b):
    src0 = edge_index[0].astype(jnp.int32)
    src_p = (src0 + (AR - QN) * (src0 // QN)).reshape(NS, PNB, PBLK)
    dst = edge_index[1].astype(jnp.int32).reshape(NS, PNB, PBLK)
    eidL, srcL, dstL = _partition(src_p, dst)

    ad = x.shape[1]
    xp = jnp.pad(x.reshape(4, QN, ad),
                 ((0, 0), (0, AR - QN), (0, 0))).reshape(PN, ad)
    h = _encode(xp, W_atom, b_atom, 1568)
    e = _encode(edge_attr, W_edge, b_edge, 8000)

    for i in range(L):
        aggp = _layer_agg(h, e, eidL, srcL, dstL)
        h = _mlp(h, aggp.reshape(PN, H), 1.0 + eps[i],
                 W1[i], b1[i], W2[i], b2[i])

    bpq = jnp.pad(batch.astype(jnp.int32).reshape(4, QN),
                  ((0, 0), (0, AR - QN)), constant_values=NG).reshape(PN)
    hp = jnp.pad(h, ((0, PPN - PN), (0, 0))).reshape(32, PJ, 128, H)
    bp = jnp.pad(bpq, (0, PPN - PN), constant_values=NG).reshape(32, PJ, 128)
    psum, pcnt = _pool(hp, bp)
    return _head(psum, pcnt, Wo1, bo1, Wo2, bo2, ln_g, ln_b)
